# baseline (speedup 1.0000x reference)
import jax
import jax.numpy as jnp
from jax.experimental import pallas as pl

_N = 1024
_TD = 64
_NB = 22
_MAXD = 40.0
_ROWS = 8


def _body(a_ref, cT_ref, ecol_ref, delta_ref, base_ref, out_ref):
    a = a_ref[...]
    cT = cT_ref[...]
    ecol = ecol_ref[...]
    delta = delta_ref[...]
    base = base_ref[...]

    dx = a[:, 0:1] - cT[0:1, :]
    dy = a[:, 1:2] - cT[1:2, :]
    dz = a[:, 2:3] - cT[2:3, :]
    d2 = dx * dx + dy * dy + dz * dz
    dist = jnp.sqrt(d2 + 1e-8)

    for r in range(_ROWS):
        dr = dist[r : r + 1, :]
        t = (dr > ecol).astype(jnp.float32)
        fr = jax.lax.dot_general(
            t, delta,
            dimension_numbers=(((0,), (0,)), ((), ())),
            preferred_element_type=jnp.float32,
        )
        out_ref[r] = fr + base


def kernel(coords, W, b):
    bin_width = _MAXD / (_NB - 1)
    edges = jnp.arange(0.0, _MAXD + bin_width, bin_width, dtype=jnp.float32)[:_NB]
    ecol = edges[: _NB - 1].reshape(_NB - 1, 1)

    table = W.T + b[None, :]
    delta = table[1:, :] - table[:-1, :]
    base = table[0:1, :]
    cT = coords.T

    grid = (_N // _ROWS,)
    return pl.pallas_call(
        _body,
        grid=grid,
        in_specs=[
            pl.BlockSpec((_ROWS, 3), lambda i: (i, 0)),
            pl.BlockSpec((3, _N), lambda i: (0, 0)),
            pl.BlockSpec((_NB - 1, 1), lambda i: (0, 0)),
            pl.BlockSpec((_NB - 1, _TD), lambda i: (0, 0)),
            pl.BlockSpec((1, _TD), lambda i: (0, 0)),
        ],
        out_specs=pl.BlockSpec((_ROWS, _N, _TD), lambda i: (i, 0, 0)),
        out_shape=jax.ShapeDtypeStruct((_N, _N, _TD), jnp.float32),
    )(coords, cT, ecol, delta, base)
